# baseline (speedup 1.0000x reference)
import functools

import jax
import jax.numpy as jnp
from jax.experimental import pallas as pl
from jax.experimental.pallas import tpu as pltpu

BEAM = 4
VOCAB = 32768
PAD_ID = 0
EOS_ID = 2
LPF = 0.6
BATCH = 128

ROWS = BATCH * BEAM
ROW_BLK = 64
RG = 16
LANES = 128
CHUNKS = VOCAB // LANES
NEG = -3.0e38
BIGIDX = 2 ** 30
NSEL = 6


def _sel_kernel(p_ref, stop_ref, wl_ref, wp_ref, score_ref, flat_ref,
                kscr, iscr):
    lane1 = jax.lax.broadcasted_iota(jnp.int32, (1, LANES), 1)
    colbad = (lane1 == PAD_ID) | (lane1 == EOS_ID)

    HALF = CHUNKS // 2

    def chain_update(key, ci, st):
        v1, v2, v3, v4, i1, i2, i3, i4 = st
        m = key > v1
        nv1 = jnp.maximum(v1, key)
        key = jnp.minimum(v1, key)
        i1, ci = jnp.where(m, ci, i1), jnp.where(m, i1, ci)
        m = key > v2
        nv2 = jnp.maximum(v2, key)
        key = jnp.minimum(v2, key)
        i2, ci = jnp.where(m, ci, i2), jnp.where(m, i2, ci)
        m = key > v3
        nv3 = jnp.maximum(v3, key)
        key = jnp.minimum(v3, key)
        i3, ci = jnp.where(m, ci, i3), jnp.where(m, i3, ci)
        m = key > v4
        nv4 = jnp.maximum(v4, key)
        i4 = jnp.where(m, ci, i4)
        return nv1, nv2, nv3, nv4, i1, i2, i3, i4

    for rg in range(ROW_BLK // RG):
        rows = pl.ds(rg * RG, RG)
        p = p_ref[rows, :]

        key0 = p * wp_ref[rows, pl.ds(0, LANES)]
        key0 = jnp.where(colbad, jnp.float32(-1.0), key0)
        keyh = p * wp_ref[rows, pl.ds(HALF * LANES, LANES)]
        lo = jnp.full((RG, LANES), jnp.float32(-2.0))
        zi = jnp.zeros((RG, LANES), jnp.int32)
        zh = jnp.full((RG, LANES), jnp.int32(HALF))

        def chunk(c, carry):
            sa = carry[:8]
            sb = carry[8:]
            xa = wp_ref[rows, pl.ds(c * LANES, LANES)]
            xb = wp_ref[rows, pl.ds((HALF + c) * LANES, LANES)]
            sa = chain_update(p * xa, c, sa)
            sb = chain_update(p * xb, HALF + c, sb)
            return sa + sb

        st = jax.lax.fori_loop(
            1, HALF, chunk,
            (key0, lo, lo, lo, zi, zi, zi, zi,
             keyh, lo, lo, lo, zh, zh, zh, zh),
            unroll=2)

        v1, v2, v3, v4 = (jnp.concatenate([st[j], st[8 + j]], axis=1)
                          for j in range(4))
        i1, i2, i3, i4 = (jnp.concatenate([st[4 + j], st[12 + j]], axis=1)
                          for j in range(4))
        lane2 = jax.lax.broadcasted_iota(jnp.int32, (RG, 2 * LANES), 1)
        lanemod = jax.lax.rem(lane2, jnp.int32(LANES))

        for t in range(NSEL):
            full1 = i1 * LANES + lanemod
            mx = jnp.max(v1, axis=1, keepdims=True)
            eq = v1 == mx
            mi = jnp.min(jnp.where(eq, full1, BIGIDX), axis=1, keepdims=True)
            sel = eq & (full1 == mi)
            kscr[rows, t:t + 1] = mx
            iscr[rows, t:t + 1] = mi
            v1 = jnp.where(sel, v2, v1)
            i1 = jnp.where(sel, i2, i1)
            v2 = jnp.where(sel, v3, v2)
            i2 = jnp.where(sel, i3, i2)
            v3 = jnp.where(sel, v4, v3)
            i3 = jnp.where(sel, i4, i3)
            v4 = jnp.where(sel, NEG, v4)

    p = p_ref[...]
    stopb = stop_ref[...] != 0

    row = jax.lax.broadcasted_iota(jnp.int32, (ROW_BLK, 1), 0)
    beam = row % BEAM
    base = beam * VOCAB

    k = kscr[...]
    vi = iscr[...]
    lane6 = jax.lax.broadcasted_iota(jnp.int32, (ROW_BLK, NSEL), 1)
    stop_f = jnp.where(lane6 < BEAM, base + lane6 + 1, BIGIDX + base + lane6)
    sel_k = jnp.where(stopb, 0.0, k)
    sel_f = jnp.where(stopb, stop_f, base + vi)

    w0 = wp_ref[:, PAD_ID:PAD_ID + 1]
    w2 = wp_ref[:, EOS_ID:EOS_ID + 1]
    k0 = jnp.where(stopb, p, p * w0)
    f0 = base
    slot7_k = jnp.where(stopb, 0.0, p * w2)
    slot7_f = jnp.where(stopb, BIGIDX + base + 7, base + EOS_ID)

    score_ref[...] = jnp.concatenate([sel_k, k0, slot7_k], axis=1)
    flat_ref[...] = jnp.concatenate([sel_f, f0, slot7_f], axis=1)


def _merge_kernel(score_ref, flat_ref, bs_ref, nw_ref, pi_ref):
    s = score_ref[...]
    f = flat_ref[...]
    batch = jax.lax.broadcasted_iota(jnp.int32, (BATCH, 1), 0)
    bs, nw, pi = [], [], []
    for _ in range(BEAM):
        mx = jnp.max(s, axis=1, keepdims=True)
        eq = s == mx
        mi = jnp.min(jnp.where(eq, f, jnp.int32(2 ** 31 - 1)),
                     axis=1, keepdims=True)
        sel = eq & (f == mi)
        bs.append(mx)
        nw.append(mi % VOCAB)
        pi.append(batch * BEAM + mi // VOCAB)
        s = jnp.where(sel, NEG, s)
    bs_ref[...] = jnp.concatenate(bs, axis=1)
    nw_ref[...] = jnp.concatenate(nw, axis=1)
    pi_ref[...] = jnp.concatenate(pi, axis=1)


@jax.jit
def kernel(word_prob, prob, stops, word_length):
    p2 = prob.reshape(ROWS, 1)
    st2 = stops.reshape(ROWS, 1)
    wl2 = word_length.reshape(ROWS, 1)

    grid = ROWS // ROW_BLK
    keys, flats = pl.pallas_call(
        _sel_kernel,
        grid=(grid,),
        in_specs=[
            pl.BlockSpec((ROW_BLK, 1), lambda i: (i, 0)),
            pl.BlockSpec((ROW_BLK, 1), lambda i: (i, 0)),
            pl.BlockSpec((ROW_BLK, 1), lambda i: (i, 0)),
            pl.BlockSpec((ROW_BLK, VOCAB), lambda i: (i, 0)),
        ],
        out_specs=[
            pl.BlockSpec((ROW_BLK, 8), lambda i: (i, 0)),
            pl.BlockSpec((ROW_BLK, 8), lambda i: (i, 0)),
        ],
        out_shape=[
            jax.ShapeDtypeStruct((ROWS, 8), jnp.float32),
            jax.ShapeDtypeStruct((ROWS, 8), jnp.int32),
        ],
        scratch_shapes=[
            pltpu.VMEM((ROW_BLK, NSEL), jnp.float32),
            pltpu.VMEM((ROW_BLK, NSEL), jnp.int32),
        ],
    )(p2, st2, wl2, word_prob)

    slot_is_sel = (jnp.arange(8, dtype=jnp.int32) < NSEL).astype(jnp.int32)
    addl = slot_is_sel[None, :] * (1 - st2)
    wl_c = wl2 + addl
    lp = (jnp.power((wl_c + 5).astype(jnp.float32), LPF)
          / jnp.power(jnp.float32(6.0), LPF))
    scores = jnp.log(jnp.clip(keys, 1e-20, 1.0)) / lp

    sc = scores.reshape(BATCH, BEAM * 8)
    fl = flats.reshape(BATCH, BEAM * 8)
    bs, nw, pi = pl.pallas_call(
        _merge_kernel,
        out_shape=[
            jax.ShapeDtypeStruct((BATCH, BEAM), jnp.float32),
            jax.ShapeDtypeStruct((BATCH, BEAM), jnp.int32),
            jax.ShapeDtypeStruct((BATCH, BEAM), jnp.int32),
        ],
    )(sc, fl)
    return bs, nw.reshape(-1), pi.reshape(-1)


# speedup vs baseline: 1.3665x; 1.3665x over previous
import functools

import jax
import jax.numpy as jnp
from jax.experimental import pallas as pl
from jax.experimental.pallas import tpu as pltpu

BEAM = 4
VOCAB = 32768
PAD_ID = 0
EOS_ID = 2
LPF = 0.6
BATCH = 128

ROWS = BATCH * BEAM
ROW_BLK = 64
RG = 32
LANES = 128
CHUNKS = VOCAB // LANES
NEG = -3.0e38
BIGIDX = 2 ** 30
NSEL = 6


def _sel_kernel(p_ref, stop_ref, wl_ref, wp_ref, score_ref, flat_ref,
                kscr, iscr):
    lane1 = jax.lax.broadcasted_iota(jnp.int32, (1, LANES), 1)
    colbad = (lane1 == PAD_ID) | (lane1 == EOS_ID)

    def chain_update(key, ci, st):
        v1, v2, v3, v4, i1, i2, i3, i4 = st
        m = key > v1
        nv1 = jnp.maximum(v1, key)
        key = jnp.minimum(v1, key)
        i1, ci = jnp.where(m, ci, i1), jnp.where(m, i1, ci)
        m = key > v2
        nv2 = jnp.maximum(v2, key)
        key = jnp.minimum(v2, key)
        i2, ci = jnp.where(m, ci, i2), jnp.where(m, i2, ci)
        m = key > v3
        nv3 = jnp.maximum(v3, key)
        key = jnp.minimum(v3, key)
        i3, ci = jnp.where(m, ci, i3), jnp.where(m, i3, ci)
        m = key > v4
        nv4 = jnp.maximum(v4, key)
        i4 = jnp.where(m, ci, i4)
        return nv1, nv2, nv3, nv4, i1, i2, i3, i4

    for rg in range(ROW_BLK // RG):
        rows = pl.ds(rg * RG, RG)
        p = p_ref[rows, :]

        key0 = p * wp_ref[rows, pl.ds(0, LANES)]
        key0 = jnp.where(colbad, jnp.float32(-1.0), key0)
        lo = jnp.full((RG, LANES), jnp.float32(-2.0))
        zi = jnp.zeros((RG, LANES), jnp.int32)

        def chunk(c, carry):
            x = wp_ref[rows, pl.ds(c * LANES, LANES)]
            return chain_update(p * x, c, carry)

        st = jax.lax.fori_loop(
            1, CHUNKS, chunk,
            (key0, lo, lo, lo, zi, zi, zi, zi),
            unroll=4)

        v1, v2, v3, v4, i1, i2, i3, i4 = st
        lanemod = jax.lax.broadcasted_iota(jnp.int32, (RG, LANES), 1)

        for t in range(NSEL):
            full1 = i1 * LANES + lanemod
            mx = jnp.max(v1, axis=1, keepdims=True)
            eq = v1 == mx
            mi = jnp.min(jnp.where(eq, full1, BIGIDX), axis=1, keepdims=True)
            sel = eq & (full1 == mi)
            kscr[rows, t:t + 1] = mx
            iscr[rows, t:t + 1] = mi
            v1 = jnp.where(sel, v2, v1)
            i1 = jnp.where(sel, i2, i1)
            v2 = jnp.where(sel, v3, v2)
            i2 = jnp.where(sel, i3, i2)
            v3 = jnp.where(sel, v4, v3)
            i3 = jnp.where(sel, i4, i3)
            v4 = jnp.where(sel, NEG, v4)

    p = p_ref[...]
    stopb = stop_ref[...] != 0

    row = jax.lax.broadcasted_iota(jnp.int32, (ROW_BLK, 1), 0)
    beam = row % BEAM
    base = beam * VOCAB

    k = kscr[...]
    vi = iscr[...]
    lane6 = jax.lax.broadcasted_iota(jnp.int32, (ROW_BLK, NSEL), 1)
    stop_f = jnp.where(lane6 < BEAM, base + lane6 + 1, BIGIDX + base + lane6)
    sel_k = jnp.where(stopb, 0.0, k)
    sel_f = jnp.where(stopb, stop_f, base + vi)

    w0 = wp_ref[:, PAD_ID:PAD_ID + 1]
    w2 = wp_ref[:, EOS_ID:EOS_ID + 1]
    k0 = jnp.where(stopb, p, p * w0)
    f0 = base
    slot7_k = jnp.where(stopb, 0.0, p * w2)
    slot7_f = jnp.where(stopb, BIGIDX + base + 7, base + EOS_ID)

    score_ref[...] = jnp.concatenate([sel_k, k0, slot7_k], axis=1)
    flat_ref[...] = jnp.concatenate([sel_f, f0, slot7_f], axis=1)


def _merge_kernel(score_ref, flat_ref, bs_ref, nw_ref, pi_ref):
    s = score_ref[...]
    f = flat_ref[...]
    batch = jax.lax.broadcasted_iota(jnp.int32, (BATCH, 1), 0)
    bs, nw, pi = [], [], []
    for _ in range(BEAM):
        mx = jnp.max(s, axis=1, keepdims=True)
        eq = s == mx
        mi = jnp.min(jnp.where(eq, f, jnp.int32(2 ** 31 - 1)),
                     axis=1, keepdims=True)
        sel = eq & (f == mi)
        bs.append(mx)
        nw.append(mi % VOCAB)
        pi.append(batch * BEAM + mi // VOCAB)
        s = jnp.where(sel, NEG, s)
    bs_ref[...] = jnp.concatenate(bs, axis=1)
    nw_ref[...] = jnp.concatenate(nw, axis=1)
    pi_ref[...] = jnp.concatenate(pi, axis=1)


@jax.jit
def kernel(word_prob, prob, stops, word_length):
    p2 = prob.reshape(ROWS, 1)
    st2 = stops.reshape(ROWS, 1)
    wl2 = word_length.reshape(ROWS, 1)

    grid = ROWS // ROW_BLK
    keys, flats = pl.pallas_call(
        _sel_kernel,
        grid=(grid,),
        in_specs=[
            pl.BlockSpec((ROW_BLK, 1), lambda i: (i, 0)),
            pl.BlockSpec((ROW_BLK, 1), lambda i: (i, 0)),
            pl.BlockSpec((ROW_BLK, 1), lambda i: (i, 0)),
            pl.BlockSpec((ROW_BLK, VOCAB), lambda i: (i, 0)),
        ],
        out_specs=[
            pl.BlockSpec((ROW_BLK, 8), lambda i: (i, 0)),
            pl.BlockSpec((ROW_BLK, 8), lambda i: (i, 0)),
        ],
        out_shape=[
            jax.ShapeDtypeStruct((ROWS, 8), jnp.float32),
            jax.ShapeDtypeStruct((ROWS, 8), jnp.int32),
        ],
        scratch_shapes=[
            pltpu.VMEM((ROW_BLK, NSEL), jnp.float32),
            pltpu.VMEM((ROW_BLK, NSEL), jnp.int32),
        ],
    )(p2, st2, wl2, word_prob)

    slot_is_sel = (jnp.arange(8, dtype=jnp.int32) < NSEL).astype(jnp.int32)
    addl = slot_is_sel[None, :] * (1 - st2)
    wl_c = wl2 + addl
    lp = (jnp.power((wl_c + 5).astype(jnp.float32), LPF)
          / jnp.power(jnp.float32(6.0), LPF))
    scores = jnp.log(jnp.clip(keys, 1e-20, 1.0)) / lp

    sc = scores.reshape(BATCH, BEAM * 8)
    fl = flats.reshape(BATCH, BEAM * 8)
    bs, nw, pi = pl.pallas_call(
        _merge_kernel,
        out_shape=[
            jax.ShapeDtypeStruct((BATCH, BEAM), jnp.float32),
            jax.ShapeDtypeStruct((BATCH, BEAM), jnp.int32),
            jax.ShapeDtypeStruct((BATCH, BEAM), jnp.int32),
        ],
    )(sc, fl)
    return bs, nw.reshape(-1), pi.reshape(-1)
